# baseline (speedup 1.0000x reference)
import functools

import jax
import jax.numpy as jnp
from jax import lax
from jax.experimental import pallas as pl
from jax.experimental.pallas import tpu as pltpu
from jax.experimental.pallas import tpu_sc as plsc

B = 4
L = 8192
D = 768


@functools.cache
def _build_sc_kernel():
    info = plsc.get_sparse_core_info()
    nw = info.num_cores * info.num_subcores
    rows_per_w = L // nw

    mesh = plsc.VectorSubcoreMesh(core_axis_name="c", subcore_axis_name="s")

    @functools.partial(
        pl.kernel,
        mesh=mesh,
        out_type=jax.ShapeDtypeStruct((B, L, D), jnp.float32),
        scratch_types=[pltpu.SemaphoreType.DMA],
    )
    def k(emb_hbm, out_hbm, sem):
        wid = lax.axis_index("s") * info.num_cores + lax.axis_index("c")
        base = wid * rows_per_w
        cps = [
            pltpu.make_async_copy(
                emb_hbm.at[pl.ds(base, rows_per_w)],
                out_hbm.at[b, pl.ds(base, rows_per_w)],
                sem,
            )
            for b in range(B)
        ]
        for cp in cps:
            cp.start()
        for cp in cps:
            cp.wait()

    return k


def kernel(x, Embed):
    return _build_sc_kernel()(Embed)


# speedup vs baseline: 50.6905x; 50.6905x over previous
import functools

import jax
import jax.numpy as jnp
from jax import lax
from jax.experimental import pallas as pl
from jax.experimental.pallas import tpu as pltpu
from jax.experimental.pallas import tpu_sc as plsc

B = 4
L = 8192
D = 768
SIZES = (16, 80, 80, 80)
OFFS = (0, 16, 96, 176)


@functools.cache
def _build_sc_kernel():
    info = plsc.get_sparse_core_info()
    nw = info.num_cores * info.num_subcores
    rows_per_w = L // nw
    assert sum(SIZES) == rows_per_w

    mesh = plsc.VectorSubcoreMesh(core_axis_name="c", subcore_axis_name="s")

    @functools.partial(
        pl.kernel,
        mesh=mesh,
        out_type=jax.ShapeDtypeStruct((B, L, D), jnp.float32),
        scratch_types=[
            pltpu.VMEM((2, max(SIZES), D), jnp.float32),
            pltpu.SemaphoreType.DMA,
        ],
    )
    def k(emb_hbm, out_hbm, buf, lsem):
        wid = lax.axis_index("s") * info.num_cores + lax.axis_index("c")
        base = wid * rows_per_w

        def load(c):
            cp = pltpu.make_async_copy(
                emb_hbm.at[pl.ds(base + OFFS[c], SIZES[c])],
                buf.at[c % 2, pl.ds(0, SIZES[c])],
                lsem,
            )
            cp.start()
            return cp

        pending = load(0)
        for c in range(len(SIZES)):
            pending.wait()
            if c + 1 < len(SIZES):
                pending = load(c + 1)
            row = base + OFFS[c]
            for b in range(B):
                pltpu.sync_copy(
                    buf.at[c % 2, pl.ds(0, SIZES[c])],
                    out_hbm.at[b, pl.ds(row, SIZES[c])],
                )

    return k


def kernel(x, Embed):
    return _build_sc_kernel()(Embed)


# speedup vs baseline: 51.9524x; 1.0249x over previous
import functools

import jax
import jax.numpy as jnp
from jax import lax
from jax.experimental import pallas as pl
from jax.experimental.pallas import tpu as pltpu
from jax.experimental.pallas import tpu_sc as plsc

B = 4
L = 8192
D = 768
CHUNK = 128


@functools.cache
def _build_sc_kernel():
    info = plsc.get_sparse_core_info()
    nw = info.num_cores * info.num_subcores
    rows_per_w = L // nw
    n_chunks = rows_per_w // CHUNK

    mesh = plsc.VectorSubcoreMesh(core_axis_name="c", subcore_axis_name="s")

    @functools.partial(
        pl.kernel,
        mesh=mesh,
        out_type=jax.ShapeDtypeStruct((B, L, D), jnp.float32),
        scratch_types=[pltpu.VMEM((CHUNK, D), jnp.float32)],
    )
    def k(emb_hbm, out_hbm, buf):
        wid = lax.axis_index("s") * info.num_cores + lax.axis_index("c")
        base = wid * rows_per_w
        for c in range(n_chunks):
            row = base + c * CHUNK
            pltpu.sync_copy(emb_hbm.at[pl.ds(row, CHUNK)], buf)
            for b in range(B):
                pltpu.sync_copy(buf, out_hbm.at[b, pl.ds(row, CHUNK)])

    return k


def kernel(x, Embed):
    return _build_sc_kernel()(Embed)
